# baseline (speedup 1.0000x reference)
import functools

import jax
import jax.numpy as jnp
from jax import lax
from jax.experimental import pallas as pl
from jax.experimental.pallas import tpu as pltpu
from jax.experimental.pallas import tpu_sc as plsc

N = 10000
E = 320000
DF = 128
DE = 16
H = 128
L = 6
G = 64

CHUNK = 80
NCHUNKS = E // CHUNK
NCORES = 2
NSUB = 16
CH_PER_CORE = NCHUNKS // NCORES
CH_PER_TILE = CH_PER_CORE // NSUB
NPAD = 10240
ROWS_PER_TILE = NPAD // NSUB

_F32 = jnp.float32


def _proj_body(nf_ref, wp_ref, bp_ref, a_ref, h_ref, p_ref):
    h = jnp.maximum(
        jnp.dot(nf_ref[...], wp_ref[...], preferred_element_type=_F32)
        + bp_ref[...][None, :], 0.0)
    h_ref[...] = h
    p_ref[...] = jnp.dot(h, a_ref[...], preferred_element_type=_F32)


def _proj(node_feats, w_proj, b_proj, a0):
    blk = 1000
    grid = N // blk
    return pl.pallas_call(
        _proj_body,
        grid=(grid,),
        in_specs=[
            pl.BlockSpec((blk, DF), lambda i: (i, 0)),
            pl.BlockSpec((DF, H), lambda i: (0, 0)),
            pl.BlockSpec((H,), lambda i: (0,)),
            pl.BlockSpec((H, H), lambda i: (0, 0)),
        ],
        out_specs=[
            pl.BlockSpec((blk, H), lambda i: (i, 0)),
            pl.BlockSpec((blk, H), lambda i: (i, 0)),
        ],
        out_shape=[
            jax.ShapeDtypeStruct((N, H), _F32),
            jax.ShapeDtypeStruct((N, H), _F32),
        ],
    )(node_feats, w_proj, b_proj, a0)


def _q_body(ef_ref, b_ref, bm_ref, q_ref):
    q_ref[...] = (
        jnp.dot(ef_ref[...], b_ref[...], preferred_element_type=_F32)
        + bm_ref[...])


def _q_one(edge_feats, b_l, bm_l):
    blk = 8000
    grid_e = E // blk
    return pl.pallas_call(
        _q_body,
        grid=(grid_e,),
        in_specs=[
            pl.BlockSpec((blk, DE), lambda e: (e, 0)),
            pl.BlockSpec((DE, H), lambda e: (0, 0)),
            pl.BlockSpec((1, H), lambda e: (0, 0)),
        ],
        out_specs=pl.BlockSpec((blk, H), lambda e: (e, 0)),
        out_shape=jax.ShapeDtypeStruct((E, H), _F32),
    )(edge_feats, b_l, bm_l)


def _update_body(h_ref, m0_ref, m1_ref, u_ref, v_ref, b_ref, a_ref,
                 hn_ref, p_ref):
    msum = m0_ref[...] + m1_ref[...]
    hn = jnp.maximum(
        jnp.dot(h_ref[...], u_ref[...], preferred_element_type=_F32)
        + jnp.dot(msum, v_ref[...], preferred_element_type=_F32)
        + b_ref[...][None, :], 0.0)
    hn_ref[...] = hn
    p_ref[...] = jnp.dot(hn, a_ref[...], preferred_element_type=_F32)


def _update(h, m_part, u, v, b, a_next):
    blk = 1000
    grid = N // blk
    m0 = m_part[:N]
    m1 = m_part[NPAD:NPAD + N]
    return pl.pallas_call(
        _update_body,
        grid=(grid,),
        in_specs=[
            pl.BlockSpec((blk, H), lambda i: (i, 0)),
            pl.BlockSpec((blk, H), lambda i: (i, 0)),
            pl.BlockSpec((blk, H), lambda i: (i, 0)),
            pl.BlockSpec((H, H), lambda i: (0, 0)),
            pl.BlockSpec((H, H), lambda i: (0, 0)),
            pl.BlockSpec((H,), lambda i: (0,)),
            pl.BlockSpec((H, H), lambda i: (0, 0)),
        ],
        out_specs=[
            pl.BlockSpec((blk, H), lambda i: (i, 0)),
            pl.BlockSpec((blk, H), lambda i: (i, 0)),
        ],
        out_shape=[
            jax.ShapeDtypeStruct((N, H), _F32),
            jax.ShapeDtypeStruct((N, H), _F32),
        ],
    )(h, m0, m1, u, v, b, a_next)


def _update_last_body(h_ref, m0_ref, m1_ref, u_ref, v_ref, b_ref, hn_ref):
    msum = m0_ref[...] + m1_ref[...]
    hn_ref[...] = jnp.maximum(
        jnp.dot(h_ref[...], u_ref[...], preferred_element_type=_F32)
        + jnp.dot(msum, v_ref[...], preferred_element_type=_F32)
        + b_ref[...][None, :], 0.0)


def _update_last(h, m_part, u, v, b):
    blk = 1000
    grid = N // blk
    return pl.pallas_call(
        _update_last_body,
        grid=(grid,),
        in_specs=[
            pl.BlockSpec((blk, H), lambda i: (i, 0)),
            pl.BlockSpec((blk, H), lambda i: (i, 0)),
            pl.BlockSpec((blk, H), lambda i: (i, 0)),
            pl.BlockSpec((H, H), lambda i: (0, 0)),
            pl.BlockSpec((H, H), lambda i: (0, 0)),
            pl.BlockSpec((H,), lambda i: (0,)),
        ],
        out_specs=pl.BlockSpec((blk, H), lambda i: (i, 0)),
        out_shape=jax.ShapeDtypeStruct((N, H), _F32),
    )(h, m_part[:N], m_part[NPAD:NPAD + N], u, v, b)


_RBLK = 1000


def _readout_body(h_ref, wa_ref, ba_ref, gid_ref, out_ref, acc_ref):
    i = pl.program_id(0)
    hb = h_ref[...]
    gidb = gid_ref[...]
    logit = jnp.sum(hb * wa_ref[...], axis=1, keepdims=True) + ba_ref[0, 0]
    whb = hb / (1.0 + jnp.exp(-logit))

    @pl.when(i == 0)
    def _():
        acc_ref[...] = jnp.concatenate(
            [jnp.zeros((G, H), _F32), jnp.full((G, H), -jnp.inf, _F32)],
            axis=1)

    row_ids = lax.broadcasted_iota(jnp.int32, (G, 1), 0)

    def body(g, _):
        mask = gidb == g
        s = jnp.sum(jnp.where(mask, whb, 0.0), axis=0)
        m = jnp.max(jnp.where(mask, hb, -jnp.inf), axis=0)
        hit = row_ids == g
        cur = acc_ref[...]
        acc_ref[...] = jnp.concatenate(
            [cur[:, :H] + jnp.where(hit, s[None, :], 0.0),
             jnp.maximum(cur[:, H:], jnp.where(hit, m[None, :], -jnp.inf))],
            axis=1)
        return 0

    lax.fori_loop(jnp.min(gidb), jnp.max(gidb) + 1, body, 0)

    @pl.when(i == pl.num_programs(0) - 1)
    def _():
        out_ref[...] = acc_ref[...]


def _readout(h, wa_row, ba, gid2d):
    return pl.pallas_call(
        _readout_body,
        grid=(N // _RBLK,),
        in_specs=[
            pl.BlockSpec((_RBLK, H), lambda i: (i, 0)),
            pl.BlockSpec((1, H), lambda i: (0, 0)),
            pl.BlockSpec((1, 1), lambda i: (0, 0)),
            pl.BlockSpec((_RBLK, 1), lambda i: (i, 0)),
        ],
        out_specs=pl.BlockSpec((G, 2 * H), lambda i: (0, 0)),
        out_shape=jax.ShapeDtypeStruct((G, 2 * H), _F32),
        scratch_shapes=[pltpu.VMEM((G, 2 * H), _F32)],
    )(h, wa_row, ba, gid2d)


def _edge_body(p_hbm, q_hbm, idx_hbm, zeros_hbm, out_hbm,
               idx0, idx1, sidx0, sidx1, rows0, rows1, q0, q1, msum_s,
               isem, qsem, gsem, ssem):
    c = lax.axis_index("c")
    s = lax.axis_index("s")
    idx = [idx0, idx1]
    sidx = [sidx0, sidx1]
    rows = [rows0, rows1]
    q = [q0, q1]

    pltpu.sync_copy(zeros_hbm.at[pl.ds(s * ROWS_PER_TILE, ROWS_PER_TILE)],
                    msum_s.at[pl.ds(s * ROWS_PER_TILE, ROWS_PER_TILE)])
    plsc.subcore_barrier()

    n = CH_PER_TILE
    ch0 = c * CH_PER_CORE + s

    def chunk_of(j):
        return ch0 + j * NSUB

    def issue_idx_q(j, b):
        pltpu.async_copy(idx_hbm.at[chunk_of(j)], idx[b], isem.at[b])
        pltpu.async_copy(q_hbm.at[pl.ds(chunk_of(j) * CHUNK, CHUNK)],
                         q[b], qsem.at[b])

    def wait_idx(j, b):
        pltpu.make_async_copy(idx_hbm.at[chunk_of(j)], idx[b],
                              isem.at[b]).wait()

    def wait_q(j, b):
        pltpu.make_async_copy(q_hbm.at[pl.ds(chunk_of(j) * CHUNK, CHUNK)],
                              q[b], qsem.at[b]).wait()

    def issue_gather(b):
        pltpu.async_copy(p_hbm.at[idx[b].at[0]], rows[b], gsem.at[b])

    def wait_gather(b):
        pltpu.make_async_copy(p_hbm.at[idx[b].at[0]], rows[b],
                              gsem.at[b]).wait()

    def copy_sidx(b):
        for k in range(CHUNK // 16):
            sl = pl.ds(k * 16, 16)
            sidx[b][sl] = idx[b][1, sl]

    def drain_scatter(b):
        pltpu.make_async_copy(p_hbm.at[pl.ds(0, CHUNK)], rows[b],
                              ssem.at[b]).wait()

    def compute(b):
        def rbody(r, _):
            for k in range(H // 16):
                sl = pl.ds(k * 16, 16)
                rows[b][r, sl] = jnp.maximum(
                    rows[b][r, sl] + q[b][r, sl], 0.0)
            return 0

        lax.fori_loop(0, CHUNK, rbody, 0)

    def process(j, b, drain=True, gather_next=True, prefetch=True):
        nb = b ^ 1
        wait_gather(b)
        wait_q(j, b)
        if gather_next:
            wait_idx(j + 1, nb)
            if drain:
                drain_scatter(nb)
            issue_gather(nb)
            copy_sidx(nb)
        compute(b)
        if prefetch:
            issue_idx_q(j + 2, b)
        pltpu.async_copy(rows[b], msum_s.at[sidx[b]], ssem.at[b], add=True)

    issue_idx_q(0, 0)
    wait_idx(0, 0)
    issue_gather(0)
    copy_sidx(0)
    issue_idx_q(1, 1)

    process(0, 0, drain=False)

    def pair(t, _):
        j = 2 * t - 1
        process(j, 1)
        process(j + 1, 0)
        return 0

    lax.fori_loop(1, 62, pair, 0)
    process(123, 1, prefetch=False)
    process(124, 0, gather_next=False, prefetch=False)
    drain_scatter(0)
    drain_scatter(1)
    plsc.subcore_barrier()
    pltpu.sync_copy(msum_s.at[pl.ds(s * ROWS_PER_TILE, ROWS_PER_TILE)],
                    out_hbm.at[pl.ds(c * NPAD + s * ROWS_PER_TILE,
                                     ROWS_PER_TILE)])


@functools.partial(jax.jit, static_argnums=())
def _edge_pass(p, q_l, idx_r, zeros):
    mesh = plsc.VectorSubcoreMesh(core_axis_name="c", subcore_axis_name="s")
    k = pl.kernel(
        _edge_body,
        out_type=jax.ShapeDtypeStruct((NCORES * NPAD, H), _F32),
        mesh=mesh,
        compiler_params=pltpu.CompilerParams(use_tc_tiling_on_sc=True),
        scratch_types=[
            pltpu.VMEM((2, CHUNK), jnp.int32),
            pltpu.VMEM((2, CHUNK), jnp.int32),
            pltpu.VMEM((CHUNK,), jnp.int32),
            pltpu.VMEM((CHUNK,), jnp.int32),
            pltpu.VMEM((CHUNK, H), _F32),
            pltpu.VMEM((CHUNK, H), _F32),
            pltpu.VMEM((CHUNK, H), _F32),
            pltpu.VMEM((CHUNK, H), _F32),
            pltpu.VMEM_SHARED((NPAD, H), _F32),
            pltpu.SemaphoreType.DMA((2,)),
            pltpu.SemaphoreType.DMA((2,)),
            pltpu.SemaphoreType.DMA((2,)),
            pltpu.SemaphoreType.DMA((2,)),
        ],
    )
    return k(p, q_l, idx_r, zeros)


def kernel(node_feats, edge_feats, W_proj, b_proj, W_msg, b_msg,
           W_new, b_new, W_atom, b_atom, edge_index, graph_ids):
    a_all = W_msg[:, :H, :]
    b_all = W_msg[:, H:, :]
    u_all = W_new[:, :H, :]
    v_all = W_new[:, H:, :]

    idx_r = jnp.stack([edge_index[0].reshape(NCHUNKS, CHUNK),
                       edge_index[1].reshape(NCHUNKS, CHUNK)], axis=1)
    zeros = jnp.zeros((NPAD, H), _F32)
    gid2d = graph_ids.reshape(N, 1)
    wa_row = W_atom.reshape(1, H)
    ba = b_atom.reshape(1, 1)

    h, p = _proj(node_feats, W_proj, b_proj, a_all[0])
    qs = [_q_one(edge_feats, b_all[l], b_msg[l].reshape(1, H))
          for l in range(L)]
    for l in range(L):
        m_part = _edge_pass(p, qs[l], idx_r, zeros)
        if l + 1 < L:
            h, p = _update(h, m_part, u_all[l], v_all[l], b_new[l],
                           a_all[l + 1])
        else:
            h = _update_last(h, m_part, u_all[l], v_all[l], b_new[l])
    return _readout(h, wa_row, ba, gid2d)


# speedup vs baseline: 1.0293x; 1.0293x over previous
import functools

import jax
import jax.numpy as jnp
from jax import lax
from jax.experimental import pallas as pl
from jax.experimental.pallas import tpu as pltpu
from jax.experimental.pallas import tpu_sc as plsc

N = 10000
E = 320000
DF = 128
DE = 16
H = 128
L = 6
G = 64

CHUNK = 80
NCHUNKS = E // CHUNK
NCORES = 2
NSUB = 16
CH_PER_CORE = NCHUNKS // NCORES
CH_PER_TILE = CH_PER_CORE // NSUB
NPAD = 10240
ROWS_PER_TILE = NPAD // NSUB

_F32 = jnp.float32


def _proj_body(nf_ref, wp_ref, bp_ref, a_ref, h_ref, p_ref):
    h = jnp.maximum(
        jnp.dot(nf_ref[...], wp_ref[...], preferred_element_type=_F32)
        + bp_ref[...][None, :], 0.0)
    h_ref[...] = h
    p_ref[...] = jnp.dot(h, a_ref[...], preferred_element_type=_F32)


def _proj(node_feats, w_proj, b_proj, a0):
    blk = 1000
    grid = N // blk
    return pl.pallas_call(
        _proj_body,
        grid=(grid,),
        in_specs=[
            pl.BlockSpec((blk, DF), lambda i: (i, 0)),
            pl.BlockSpec((DF, H), lambda i: (0, 0)),
            pl.BlockSpec((H,), lambda i: (0,)),
            pl.BlockSpec((H, H), lambda i: (0, 0)),
        ],
        out_specs=[
            pl.BlockSpec((blk, H), lambda i: (i, 0)),
            pl.BlockSpec((blk, H), lambda i: (i, 0)),
        ],
        out_shape=[
            jax.ShapeDtypeStruct((N, H), _F32),
            jax.ShapeDtypeStruct((N, H), _F32),
        ],
    )(node_feats, w_proj, b_proj, a0)


def _q_body(ef_ref, b_ref, bm_ref, q_ref):
    q_ref[...] = (
        jnp.dot(ef_ref[...], b_ref[...], preferred_element_type=_F32)
        + bm_ref[...])


def _q_one(edge_feats, b_l, bm_l):
    blk = 2000
    grid_e = E // blk
    return pl.pallas_call(
        _q_body,
        grid=(grid_e,),
        in_specs=[
            pl.BlockSpec((blk, DE), lambda e: (e, 0)),
            pl.BlockSpec((DE, H), lambda e: (0, 0)),
            pl.BlockSpec((1, H), lambda e: (0, 0)),
        ],
        out_specs=pl.BlockSpec((blk, H), lambda e: (e, 0)),
        out_shape=jax.ShapeDtypeStruct((E, H), _F32),
    )(edge_feats, b_l, bm_l)


def _update_body(h_ref, m0_ref, m1_ref, u_ref, v_ref, b_ref, a_ref,
                 hn_ref, p_ref):
    msum = m0_ref[...] + m1_ref[...]
    hn = jnp.maximum(
        jnp.dot(h_ref[...], u_ref[...], preferred_element_type=_F32)
        + jnp.dot(msum, v_ref[...], preferred_element_type=_F32)
        + b_ref[...][None, :], 0.0)
    hn_ref[...] = hn
    p_ref[...] = jnp.dot(hn, a_ref[...], preferred_element_type=_F32)


def _update(h, m_part, u, v, b, a_next):
    blk = 1000
    grid = N // blk
    m0 = m_part[:N]
    m1 = m_part[NPAD:NPAD + N]
    return pl.pallas_call(
        _update_body,
        grid=(grid,),
        in_specs=[
            pl.BlockSpec((blk, H), lambda i: (i, 0)),
            pl.BlockSpec((blk, H), lambda i: (i, 0)),
            pl.BlockSpec((blk, H), lambda i: (i, 0)),
            pl.BlockSpec((H, H), lambda i: (0, 0)),
            pl.BlockSpec((H, H), lambda i: (0, 0)),
            pl.BlockSpec((H,), lambda i: (0,)),
            pl.BlockSpec((H, H), lambda i: (0, 0)),
        ],
        out_specs=[
            pl.BlockSpec((blk, H), lambda i: (i, 0)),
            pl.BlockSpec((blk, H), lambda i: (i, 0)),
        ],
        out_shape=[
            jax.ShapeDtypeStruct((N, H), _F32),
            jax.ShapeDtypeStruct((N, H), _F32),
        ],
    )(h, m0, m1, u, v, b, a_next)


def _update_last_body(h_ref, m0_ref, m1_ref, u_ref, v_ref, b_ref, hn_ref):
    msum = m0_ref[...] + m1_ref[...]
    hn_ref[...] = jnp.maximum(
        jnp.dot(h_ref[...], u_ref[...], preferred_element_type=_F32)
        + jnp.dot(msum, v_ref[...], preferred_element_type=_F32)
        + b_ref[...][None, :], 0.0)


def _update_last(h, m_part, u, v, b):
    blk = 1000
    grid = N // blk
    return pl.pallas_call(
        _update_last_body,
        grid=(grid,),
        in_specs=[
            pl.BlockSpec((blk, H), lambda i: (i, 0)),
            pl.BlockSpec((blk, H), lambda i: (i, 0)),
            pl.BlockSpec((blk, H), lambda i: (i, 0)),
            pl.BlockSpec((H, H), lambda i: (0, 0)),
            pl.BlockSpec((H, H), lambda i: (0, 0)),
            pl.BlockSpec((H,), lambda i: (0,)),
        ],
        out_specs=pl.BlockSpec((blk, H), lambda i: (i, 0)),
        out_shape=jax.ShapeDtypeStruct((N, H), _F32),
    )(h, m_part[:N], m_part[NPAD:NPAD + N], u, v, b)


_RBLK = 1000


def _readout_body(h_ref, wa_ref, ba_ref, gid_ref, out_ref, acc_ref):
    i = pl.program_id(0)
    hb = h_ref[...]
    gidb = gid_ref[...]
    logit = jnp.sum(hb * wa_ref[...], axis=1, keepdims=True) + ba_ref[0, 0]
    whb = hb / (1.0 + jnp.exp(-logit))

    @pl.when(i == 0)
    def _():
        acc_ref[...] = jnp.concatenate(
            [jnp.zeros((G, H), _F32), jnp.full((G, H), -jnp.inf, _F32)],
            axis=1)

    row_ids = lax.broadcasted_iota(jnp.int32, (G, 1), 0)

    def body(g, _):
        mask = gidb == g
        s = jnp.sum(jnp.where(mask, whb, 0.0), axis=0)
        m = jnp.max(jnp.where(mask, hb, -jnp.inf), axis=0)
        hit = row_ids == g
        cur = acc_ref[...]
        acc_ref[...] = jnp.concatenate(
            [cur[:, :H] + jnp.where(hit, s[None, :], 0.0),
             jnp.maximum(cur[:, H:], jnp.where(hit, m[None, :], -jnp.inf))],
            axis=1)
        return 0

    lax.fori_loop(jnp.min(gidb), jnp.max(gidb) + 1, body, 0)

    @pl.when(i == pl.num_programs(0) - 1)
    def _():
        out_ref[...] = acc_ref[...]


def _readout(h, wa_row, ba, gid2d):
    return pl.pallas_call(
        _readout_body,
        grid=(N // _RBLK,),
        in_specs=[
            pl.BlockSpec((_RBLK, H), lambda i: (i, 0)),
            pl.BlockSpec((1, H), lambda i: (0, 0)),
            pl.BlockSpec((1, 1), lambda i: (0, 0)),
            pl.BlockSpec((_RBLK, 1), lambda i: (i, 0)),
        ],
        out_specs=pl.BlockSpec((G, 2 * H), lambda i: (0, 0)),
        out_shape=jax.ShapeDtypeStruct((G, 2 * H), _F32),
        scratch_shapes=[pltpu.VMEM((G, 2 * H), _F32)],
    )(h, wa_row, ba, gid2d)


def _edge_body(p_hbm, q_hbm, src_hbm, dst_hbm, zeros_hbm, out_hbm,
               isrc0, isrc1, idst0, idst1, sidx0, sidx1,
               rows0, rows1, q0, q1, msum_s,
               isem, dsem, qsem, gsem, ssem):
    c = lax.axis_index("c")
    s = lax.axis_index("s")
    isrc = [isrc0, isrc1]
    idst = [idst0, idst1]
    sidx = [sidx0, sidx1]
    rows = [rows0, rows1]
    q = [q0, q1]

    pltpu.sync_copy(zeros_hbm.at[pl.ds(s * ROWS_PER_TILE, ROWS_PER_TILE)],
                    msum_s.at[pl.ds(s * ROWS_PER_TILE, ROWS_PER_TILE)])
    plsc.subcore_barrier()

    n = CH_PER_TILE
    ch0 = c * CH_PER_CORE + s

    def chunk_of(j):
        return ch0 + j * NSUB

    def issue_idx_q(j, b):
        off = pl.ds(chunk_of(j) * CHUNK, CHUNK)
        pltpu.async_copy(src_hbm.at[off], isrc[b], isem.at[b])
        pltpu.async_copy(dst_hbm.at[off], idst[b], dsem.at[b])
        pltpu.async_copy(q_hbm.at[off], q[b], qsem.at[b])

    def wait_idx(j, b):
        off = pl.ds(chunk_of(j) * CHUNK, CHUNK)
        pltpu.make_async_copy(src_hbm.at[off], isrc[b], isem.at[b]).wait()
        pltpu.make_async_copy(dst_hbm.at[off], idst[b], dsem.at[b]).wait()

    def wait_q(j, b):
        pltpu.make_async_copy(q_hbm.at[pl.ds(chunk_of(j) * CHUNK, CHUNK)],
                              q[b], qsem.at[b]).wait()

    def issue_gather(b):
        pltpu.async_copy(p_hbm.at[isrc[b]], rows[b], gsem.at[b])

    def wait_gather(b):
        pltpu.make_async_copy(p_hbm.at[isrc[b]], rows[b],
                              gsem.at[b]).wait()

    def copy_sidx(b):
        for k in range(CHUNK // 16):
            sl = pl.ds(k * 16, 16)
            sidx[b][sl] = idst[b][sl]

    def drain_scatter(b):
        pltpu.make_async_copy(p_hbm.at[pl.ds(0, CHUNK)], rows[b],
                              ssem.at[b]).wait()

    def compute(b):
        def rbody(r, _):
            for k in range(H // 16):
                sl = pl.ds(k * 16, 16)
                rows[b][r, sl] = jnp.maximum(
                    rows[b][r, sl] + q[b][r, sl], 0.0)
            return 0

        lax.fori_loop(0, CHUNK, rbody, 0)

    def process(j, b, drain=True, gather_next=True, prefetch=True):
        nb = b ^ 1
        wait_gather(b)
        wait_q(j, b)
        if gather_next:
            wait_idx(j + 1, nb)
            if drain:
                drain_scatter(nb)
            issue_gather(nb)
            copy_sidx(nb)
        compute(b)
        if prefetch:
            issue_idx_q(j + 2, b)
        pltpu.async_copy(rows[b], msum_s.at[sidx[b]], ssem.at[b], add=True)

    issue_idx_q(0, 0)
    wait_idx(0, 0)
    issue_gather(0)
    copy_sidx(0)
    issue_idx_q(1, 1)

    process(0, 0, drain=False)

    def pair(t, _):
        j = 2 * t - 1
        process(j, 1)
        process(j + 1, 0)
        return 0

    lax.fori_loop(1, 62, pair, 0)
    process(123, 1, prefetch=False)
    process(124, 0, gather_next=False, prefetch=False)
    drain_scatter(0)
    drain_scatter(1)
    plsc.subcore_barrier()
    pltpu.sync_copy(msum_s.at[pl.ds(s * ROWS_PER_TILE, ROWS_PER_TILE)],
                    out_hbm.at[pl.ds(c * NPAD + s * ROWS_PER_TILE,
                                     ROWS_PER_TILE)])


@functools.partial(jax.jit, static_argnums=())
def _edge_pass(p, q_l, src_f, dst_f, zeros):
    mesh = plsc.VectorSubcoreMesh(core_axis_name="c", subcore_axis_name="s")
    k = pl.kernel(
        _edge_body,
        out_type=jax.ShapeDtypeStruct((NCORES * NPAD, H), _F32),
        mesh=mesh,
        compiler_params=pltpu.CompilerParams(use_tc_tiling_on_sc=True),
        scratch_types=[
            pltpu.VMEM((CHUNK,), jnp.int32),
            pltpu.VMEM((CHUNK,), jnp.int32),
            pltpu.VMEM((CHUNK,), jnp.int32),
            pltpu.VMEM((CHUNK,), jnp.int32),
            pltpu.VMEM((CHUNK,), jnp.int32),
            pltpu.VMEM((CHUNK,), jnp.int32),
            pltpu.VMEM((CHUNK, H), _F32),
            pltpu.VMEM((CHUNK, H), _F32),
            pltpu.VMEM((CHUNK, H), _F32),
            pltpu.VMEM((CHUNK, H), _F32),
            pltpu.VMEM_SHARED((NPAD, H), _F32),
            pltpu.SemaphoreType.DMA((2,)),
            pltpu.SemaphoreType.DMA((2,)),
            pltpu.SemaphoreType.DMA((2,)),
            pltpu.SemaphoreType.DMA((2,)),
            pltpu.SemaphoreType.DMA((2,)),
        ],
    )
    return k(p, q_l, src_f, dst_f, zeros)


def kernel(node_feats, edge_feats, W_proj, b_proj, W_msg, b_msg,
           W_new, b_new, W_atom, b_atom, edge_index, graph_ids):
    a_all = W_msg[:, :H, :]
    b_all = W_msg[:, H:, :]
    u_all = W_new[:, :H, :]
    v_all = W_new[:, H:, :]

    src_f = edge_index[0]
    dst_f = edge_index[1]
    zeros = jnp.zeros((NPAD, H), _F32)
    gid2d = graph_ids.reshape(N, 1)
    wa_row = W_atom.reshape(1, H)
    ba = b_atom.reshape(1, 1)

    h, p = _proj(node_feats, W_proj, b_proj, a_all[0])
    qs = [_q_one(edge_feats, b_all[l], b_msg[l].reshape(1, H))
          for l in range(L)]
    for l in range(L):
        m_part = _edge_pass(p, qs[l], src_f, dst_f, zeros)
        if l + 1 < L:
            h, p = _update(h, m_part, u_all[l], v_all[l], b_new[l],
                           a_all[l + 1])
        else:
            h = _update_last(h, m_part, u_all[l], v_all[l], b_new[l])
    return _readout(h, wa_row, ba, gid2d)


# speedup vs baseline: 1.2109x; 1.1764x over previous
import functools

import jax
import jax.numpy as jnp
from jax import lax
from jax.experimental import pallas as pl
from jax.experimental.pallas import tpu as pltpu
from jax.experimental.pallas import tpu_sc as plsc

N = 10000
E = 320000
DF = 128
DE = 16
H = 128
L = 6
G = 64

CHUNK = 80
NCHUNKS = E // CHUNK
NCORES = 2
NSUB = 16
CH_PER_CORE = NCHUNKS // NCORES
CH_PER_TILE = CH_PER_CORE // NSUB
NPAD = 10240
ROWS_PER_TILE = NPAD // NSUB

_F32 = jnp.float32


def _proj_body(nf_ref, wp_ref, bp_ref, a_ref, h_ref, p_ref):
    h = jnp.maximum(
        jnp.dot(nf_ref[...], wp_ref[...], preferred_element_type=_F32)
        + bp_ref[...][None, :], 0.0)
    h_ref[...] = h
    p_ref[...] = jnp.dot(h, a_ref[...], preferred_element_type=_F32)


def _proj(node_feats, w_proj, b_proj, a0):
    blk = 1000
    grid = N // blk
    return pl.pallas_call(
        _proj_body,
        grid=(grid,),
        in_specs=[
            pl.BlockSpec((blk, DF), lambda i: (i, 0)),
            pl.BlockSpec((DF, H), lambda i: (0, 0)),
            pl.BlockSpec((H,), lambda i: (0,)),
            pl.BlockSpec((H, H), lambda i: (0, 0)),
        ],
        out_specs=[
            pl.BlockSpec((blk, H), lambda i: (i, 0)),
            pl.BlockSpec((blk, H), lambda i: (i, 0)),
        ],
        out_shape=[
            jax.ShapeDtypeStruct((N, H), _F32),
            jax.ShapeDtypeStruct((N, H), _F32),
        ],
    )(node_feats, w_proj, b_proj, a0)


def _q_body(ef_ref, b_ref, bm_ref, q_ref):
    q_ref[...] = lax.dot_general(
        ef_ref[...], b_ref[...], (((0,), (0,)), ((), ())),
        preferred_element_type=_F32) + bm_ref[...]


def _q_one(ef_t, b_l, bm_l):
    blk = 2560
    grid_e = E // blk
    return pl.pallas_call(
        _q_body,
        grid=(grid_e,),
        in_specs=[
            pl.BlockSpec((DE, blk), lambda e: (0, e)),
            pl.BlockSpec((DE, H), lambda e: (0, 0)),
            pl.BlockSpec((1, H), lambda e: (0, 0)),
        ],
        out_specs=pl.BlockSpec((blk, H), lambda e: (e, 0)),
        out_shape=jax.ShapeDtypeStruct((E, H), _F32),
    )(ef_t, b_l, bm_l)


def _update_body(h_ref, m0_ref, m1_ref, u_ref, v_ref, b_ref, a_ref,
                 hn_ref, p_ref):
    msum = m0_ref[...] + m1_ref[...]
    hn = jnp.maximum(
        jnp.dot(h_ref[...], u_ref[...], preferred_element_type=_F32)
        + jnp.dot(msum, v_ref[...], preferred_element_type=_F32)
        + b_ref[...][None, :], 0.0)
    hn_ref[...] = hn
    p_ref[...] = jnp.dot(hn, a_ref[...], preferred_element_type=_F32)


def _update(h, m_part, u, v, b, a_next):
    blk = 1000
    grid = N // blk
    m0 = m_part[:N]
    m1 = m_part[NPAD:NPAD + N]
    return pl.pallas_call(
        _update_body,
        grid=(grid,),
        in_specs=[
            pl.BlockSpec((blk, H), lambda i: (i, 0)),
            pl.BlockSpec((blk, H), lambda i: (i, 0)),
            pl.BlockSpec((blk, H), lambda i: (i, 0)),
            pl.BlockSpec((H, H), lambda i: (0, 0)),
            pl.BlockSpec((H, H), lambda i: (0, 0)),
            pl.BlockSpec((H,), lambda i: (0,)),
            pl.BlockSpec((H, H), lambda i: (0, 0)),
        ],
        out_specs=[
            pl.BlockSpec((blk, H), lambda i: (i, 0)),
            pl.BlockSpec((blk, H), lambda i: (i, 0)),
        ],
        out_shape=[
            jax.ShapeDtypeStruct((N, H), _F32),
            jax.ShapeDtypeStruct((N, H), _F32),
        ],
    )(h, m0, m1, u, v, b, a_next)


def _update_last_body(h_ref, m0_ref, m1_ref, u_ref, v_ref, b_ref, hn_ref):
    msum = m0_ref[...] + m1_ref[...]
    hn_ref[...] = jnp.maximum(
        jnp.dot(h_ref[...], u_ref[...], preferred_element_type=_F32)
        + jnp.dot(msum, v_ref[...], preferred_element_type=_F32)
        + b_ref[...][None, :], 0.0)


def _update_last(h, m_part, u, v, b):
    blk = 1000
    grid = N // blk
    return pl.pallas_call(
        _update_last_body,
        grid=(grid,),
        in_specs=[
            pl.BlockSpec((blk, H), lambda i: (i, 0)),
            pl.BlockSpec((blk, H), lambda i: (i, 0)),
            pl.BlockSpec((blk, H), lambda i: (i, 0)),
            pl.BlockSpec((H, H), lambda i: (0, 0)),
            pl.BlockSpec((H, H), lambda i: (0, 0)),
            pl.BlockSpec((H,), lambda i: (0,)),
        ],
        out_specs=pl.BlockSpec((blk, H), lambda i: (i, 0)),
        out_shape=jax.ShapeDtypeStruct((N, H), _F32),
    )(h, m_part[:N], m_part[NPAD:NPAD + N], u, v, b)


_RBLK = 1000


def _readout_body(h_ref, wa_ref, ba_ref, gid_ref, out_ref, acc_ref):
    i = pl.program_id(0)
    hb = h_ref[...]
    gidb = gid_ref[...]
    logit = jnp.sum(hb * wa_ref[...], axis=1, keepdims=True) + ba_ref[0, 0]
    whb = hb / (1.0 + jnp.exp(-logit))

    @pl.when(i == 0)
    def _():
        acc_ref[...] = jnp.concatenate(
            [jnp.zeros((G, H), _F32), jnp.full((G, H), -jnp.inf, _F32)],
            axis=1)

    row_ids = lax.broadcasted_iota(jnp.int32, (G, 1), 0)

    def body(g, _):
        mask = gidb == g
        s = jnp.sum(jnp.where(mask, whb, 0.0), axis=0)
        m = jnp.max(jnp.where(mask, hb, -jnp.inf), axis=0)
        hit = row_ids == g
        cur = acc_ref[...]
        acc_ref[...] = jnp.concatenate(
            [cur[:, :H] + jnp.where(hit, s[None, :], 0.0),
             jnp.maximum(cur[:, H:], jnp.where(hit, m[None, :], -jnp.inf))],
            axis=1)
        return 0

    lax.fori_loop(jnp.min(gidb), jnp.max(gidb) + 1, body, 0)

    @pl.when(i == pl.num_programs(0) - 1)
    def _():
        out_ref[...] = acc_ref[...]


def _readout(h, wa_row, ba, gid2d):
    return pl.pallas_call(
        _readout_body,
        grid=(N // _RBLK,),
        in_specs=[
            pl.BlockSpec((_RBLK, H), lambda i: (i, 0)),
            pl.BlockSpec((1, H), lambda i: (0, 0)),
            pl.BlockSpec((1, 1), lambda i: (0, 0)),
            pl.BlockSpec((_RBLK, 1), lambda i: (i, 0)),
        ],
        out_specs=pl.BlockSpec((G, 2 * H), lambda i: (0, 0)),
        out_shape=jax.ShapeDtypeStruct((G, 2 * H), _F32),
        scratch_shapes=[pltpu.VMEM((G, 2 * H), _F32)],
    )(h, wa_row, ba, gid2d)


def _edge_body(p_hbm, q_hbm, src_hbm, dst_hbm, zeros_hbm, out_hbm,
               isrc0, isrc1, idst0, idst1, sidx0, sidx1,
               rows0, rows1, q0, q1, msum_s,
               isem, dsem, qsem, gsem, ssem):
    c = lax.axis_index("c")
    s = lax.axis_index("s")
    isrc = [isrc0, isrc1]
    idst = [idst0, idst1]
    sidx = [sidx0, sidx1]
    rows = [rows0, rows1]
    q = [q0, q1]

    pltpu.sync_copy(zeros_hbm.at[pl.ds(s * ROWS_PER_TILE, ROWS_PER_TILE)],
                    msum_s.at[pl.ds(s * ROWS_PER_TILE, ROWS_PER_TILE)])
    plsc.subcore_barrier()

    n = CH_PER_TILE
    ch0 = c * CH_PER_CORE + s

    def chunk_of(j):
        return ch0 + j * NSUB

    def issue_idx_q(j, b):
        off = pl.ds(chunk_of(j) * CHUNK, CHUNK)
        pltpu.async_copy(src_hbm.at[off], isrc[b], isem.at[b])
        pltpu.async_copy(dst_hbm.at[off], idst[b], dsem.at[b])
        pltpu.async_copy(q_hbm.at[off], q[b], qsem.at[b])

    def wait_idx(j, b):
        off = pl.ds(chunk_of(j) * CHUNK, CHUNK)
        pltpu.make_async_copy(src_hbm.at[off], isrc[b], isem.at[b]).wait()
        pltpu.make_async_copy(dst_hbm.at[off], idst[b], dsem.at[b]).wait()

    def wait_q(j, b):
        pltpu.make_async_copy(q_hbm.at[pl.ds(chunk_of(j) * CHUNK, CHUNK)],
                              q[b], qsem.at[b]).wait()

    def issue_gather(b):
        pltpu.async_copy(p_hbm.at[isrc[b]], rows[b], gsem.at[b])

    def wait_gather(b):
        pltpu.make_async_copy(p_hbm.at[isrc[b]], rows[b],
                              gsem.at[b]).wait()

    def copy_sidx(b):
        for k in range(CHUNK // 16):
            sl = pl.ds(k * 16, 16)
            sidx[b][sl] = idst[b][sl]

    def drain_scatter(b):
        pltpu.make_async_copy(p_hbm.at[pl.ds(0, CHUNK)], rows[b],
                              ssem.at[b]).wait()

    def compute(b):
        def rbody(r, _):
            for k in range(H // 16):
                sl = pl.ds(k * 16, 16)
                rows[b][r, sl] = jnp.maximum(
                    rows[b][r, sl] + q[b][r, sl], 0.0)
            return 0

        lax.fori_loop(0, CHUNK, rbody, 0)

    def process(j, b, drain=True, gather_next=True, prefetch=True):
        nb = b ^ 1
        wait_gather(b)
        wait_q(j, b)
        if gather_next:
            wait_idx(j + 1, nb)
            if drain:
                drain_scatter(nb)
            issue_gather(nb)
            copy_sidx(nb)
        compute(b)
        if prefetch:
            issue_idx_q(j + 2, b)
        pltpu.async_copy(rows[b], msum_s.at[sidx[b]], ssem.at[b], add=True)

    issue_idx_q(0, 0)
    wait_idx(0, 0)
    issue_gather(0)
    copy_sidx(0)
    issue_idx_q(1, 1)

    process(0, 0, drain=False)

    def pair(t, _):
        j = 2 * t - 1
        process(j, 1)
        process(j + 1, 0)
        return 0

    lax.fori_loop(1, 62, pair, 0)
    process(123, 1, prefetch=False)
    process(124, 0, gather_next=False, prefetch=False)
    drain_scatter(0)
    drain_scatter(1)
    plsc.subcore_barrier()
    pltpu.sync_copy(msum_s.at[pl.ds(s * ROWS_PER_TILE, ROWS_PER_TILE)],
                    out_hbm.at[pl.ds(c * NPAD + s * ROWS_PER_TILE,
                                     ROWS_PER_TILE)])


@functools.partial(jax.jit, static_argnums=())
def _edge_pass(p, q_l, src_f, dst_f, zeros):
    mesh = plsc.VectorSubcoreMesh(core_axis_name="c", subcore_axis_name="s")
    k = pl.kernel(
        _edge_body,
        out_type=jax.ShapeDtypeStruct((NCORES * NPAD, H), _F32),
        mesh=mesh,
        compiler_params=pltpu.CompilerParams(use_tc_tiling_on_sc=True),
        scratch_types=[
            pltpu.VMEM((CHUNK,), jnp.int32),
            pltpu.VMEM((CHUNK,), jnp.int32),
            pltpu.VMEM((CHUNK,), jnp.int32),
            pltpu.VMEM((CHUNK,), jnp.int32),
            pltpu.VMEM((CHUNK,), jnp.int32),
            pltpu.VMEM((CHUNK,), jnp.int32),
            pltpu.VMEM((CHUNK, H), _F32),
            pltpu.VMEM((CHUNK, H), _F32),
            pltpu.VMEM((CHUNK, H), _F32),
            pltpu.VMEM((CHUNK, H), _F32),
            pltpu.VMEM_SHARED((NPAD, H), _F32),
            pltpu.SemaphoreType.DMA((2,)),
            pltpu.SemaphoreType.DMA((2,)),
            pltpu.SemaphoreType.DMA((2,)),
            pltpu.SemaphoreType.DMA((2,)),
            pltpu.SemaphoreType.DMA((2,)),
        ],
    )
    return k(p, q_l, src_f, dst_f, zeros)


def kernel(node_feats, edge_feats, W_proj, b_proj, W_msg, b_msg,
           W_new, b_new, W_atom, b_atom, edge_index, graph_ids):
    a_all = W_msg[:, :H, :]
    b_all = W_msg[:, H:, :]
    u_all = W_new[:, :H, :]
    v_all = W_new[:, H:, :]

    src_f = edge_index[0]
    dst_f = edge_index[1]
    zeros = jnp.zeros((NPAD, H), _F32)
    gid2d = graph_ids.reshape(N, 1)
    wa_row = W_atom.reshape(1, H)
    ba = b_atom.reshape(1, 1)

    ef_t = edge_feats.T
    h, p = _proj(node_feats, W_proj, b_proj, a_all[0])
    qs = [_q_one(ef_t, b_all[l], b_msg[l].reshape(1, H))
          for l in range(L)]
    for l in range(L):
        m_part = _edge_pass(p, qs[l], src_f, dst_f, zeros)
        if l + 1 < L:
            h, p = _update(h, m_part, u_all[l], v_all[l], b_new[l],
                           a_all[l + 1])
        else:
            h = _update_last(h, m_part, u_all[l], v_all[l], b_new[l])
    return _readout(h, wa_row, ba, gid2d)
